# baseline (speedup 1.0000x reference)
import functools

import jax
import jax.numpy as jnp
from jax import lax
from jax.experimental import pallas as pl
from jax.experimental.pallas import tpu as pltpu
from jax.experimental.pallas import tpu_sc as plsc

EMBED_DIM = 64
VOCAB_ROWS = 300
NUM_WORKERS = 32
CHUNK = 128
LANES = 16
SKEW = EMBED_DIM + 1


def _embed_kernel_body(n_chunks, idx_hbm, emb_hbm, out5_hbm,
                       idx_v, tab_sk, tbuf, wsem, isem):
    wid = lax.axis_index("s") * 2 + lax.axis_index("c")
    per_w = n_chunks * CHUNK
    k0 = wid * n_chunks
    idx_copy = pltpu.make_async_copy(
        idx_hbm.at[pl.ds(k0 * CHUNK, per_w)], idx_v, isem
    )
    idx_copy.start()
    pltpu.sync_copy(emb_hbm, tab_sk.at[:, pl.ds(0, EMBED_DIM)])
    idx_copy.wait()

    def write_descs(j, p):
        k = k0 + j
        t = k // 128
        b_hi = k % 128
        return [
            pltpu.make_async_copy(
                tbuf.at[p, pl.ds(c_hi * 8, 8)],
                out5_hbm.at[t, c_hi, b_hi],
                wsem.at[p],
            )
            for c_hi in range(8)
        ]

    def do_chunk(j):
        p = lax.rem(j, 2)
        rows = [idx_v[pl.ds(j * CHUNK + g * LANES, LANES)] for g in range(8)]
        zero = jnp.zeros((LANES,), jnp.int32)

        @pl.when(j >= 2)
        def _():
            for d in write_descs(j - 2, p):
                d.wait()

        @plsc.parallel_loop(0, EMBED_DIM, step=1, unroll=8)
        def _col(c):
            cv = zero + c
            for g in range(8):
                tbuf[p, c, pl.ds(g * LANES, LANES)] = plsc.load_gather(
                    tab_sk, [rows[g], cv]
                )

        for d in write_descs(j, p):
            d.start()

    def loop_body(j, carry):
        do_chunk(j)
        return carry

    lax.fori_loop(0, n_chunks, loop_body, 0)

    for j in (n_chunks - 2, n_chunks - 1):
        for d in write_descs(j, lax.rem(j, 2)):
            d.wait()


@functools.partial(jax.jit, static_argnames=("b", "t"))
def _embed(idx_tmajor, emb, b, t):
    n_chunks_total = (b // CHUNK) * t
    n_chunks = n_chunks_total // NUM_WORKERS
    per_w = n_chunks * CHUNK
    mesh = plsc.VectorSubcoreMesh(
        core_axis_name="c", subcore_axis_name="s", num_cores=2, num_subcores=16
    )
    run = pl.kernel(
        functools.partial(_embed_kernel_body, n_chunks),
        out_type=jax.ShapeDtypeStruct(
            (t, EMBED_DIM // 8, b // CHUNK, 8, CHUNK), jnp.float32
        ),
        mesh=mesh,
        scratch_types=[
            pltpu.VMEM((per_w,), jnp.int32),
            pltpu.VMEM((VOCAB_ROWS, SKEW), jnp.float32),
            pltpu.VMEM((2, EMBED_DIM, CHUNK), jnp.float32),
            pltpu.SemaphoreType.DMA((2,)),
            pltpu.SemaphoreType.DMA,
        ],
        compiler_params=pltpu.CompilerParams(
            use_tc_tiling_on_sc=False, needs_layout_passes=False
        ),
    )
    return run(idx_tmajor, emb)


def kernel(x, emb):
    b, t = x.shape
    assert emb.shape == (VOCAB_ROWS, EMBED_DIM)
    idx_tmajor = x.T.reshape(-1).astype(jnp.int32)
    y5 = _embed(idx_tmajor, emb, b, t)
    return y5.transpose(2, 4, 0, 1, 3).reshape(b, t, EMBED_DIM)


# speedup vs baseline: 1.2911x; 1.2911x over previous
import functools

import jax
import jax.numpy as jnp
from jax import lax
from jax.experimental import pallas as pl
from jax.experimental.pallas import tpu as pltpu
from jax.experimental.pallas import tpu_sc as plsc

EMBED_DIM = 64
VOCAB_ROWS = 300
NUM_WORKERS = 32
CHUNK = 128
LANES = 16
SKEW = EMBED_DIM + 1


def _embed_kernel_body(n_chunks, idx_hbm, emb_hbm, out5_hbm,
                       idx_v, tab_raw, tab_sk, tbuf, wsem, isem):
    wid = lax.axis_index("s") * 2 + lax.axis_index("c")
    per_w = n_chunks * CHUNK
    k0 = wid * n_chunks
    idx_copy = pltpu.make_async_copy(
        idx_hbm.at[pl.ds(k0 * CHUNK, per_w)], idx_v, isem
    )
    idx_copy.start()
    pltpu.sync_copy(emb_hbm, tab_raw)

    iota = lax.iota(jnp.int32, LANES)

    @plsc.parallel_loop(0, VOCAB_ROWS, step=1, unroll=4)
    def _skew(r):
        base = r * SKEW + iota
        for q in range(EMBED_DIM // LANES):
            plsc.store_scatter(
                tab_sk, [base + q * LANES], tab_raw[r, pl.ds(q * LANES, LANES)]
            )

    idx_copy.wait()

    def write_descs(j, p):
        k = k0 + j
        t = k // 128
        b_hi = k % 128
        return [
            pltpu.make_async_copy(
                tbuf.at[p, pl.ds(c_hi * 8, 8)],
                out5_hbm.at[t, c_hi, b_hi],
                wsem.at[p],
            )
            for c_hi in range(8)
        ]

    def do_chunk(j):
        p = lax.rem(j, 2)
        bases = [
            idx_v[pl.ds(j * CHUNK + g * LANES, LANES)] * SKEW for g in range(8)
        ]

        @pl.when(j >= 2)
        def _():
            for d in write_descs(j - 2, p):
                d.wait()

        @plsc.parallel_loop(0, EMBED_DIM, step=1, unroll=8)
        def _col(c):
            for g in range(8):
                tbuf[p, c, pl.ds(g * LANES, LANES)] = plsc.load_gather(
                    tab_sk, [bases[g] + c]
                )

        for d in write_descs(j, p):
            d.start()

    def loop_body(j, carry):
        do_chunk(j)
        return carry

    lax.fori_loop(0, n_chunks, loop_body, 0)

    for j in (n_chunks - 2, n_chunks - 1):
        for d in write_descs(j, lax.rem(j, 2)):
            d.wait()


@functools.partial(jax.jit, static_argnames=("b", "t"))
def _embed(idx_tmajor, emb, b, t):
    n_chunks_total = (b // CHUNK) * t
    n_chunks = n_chunks_total // NUM_WORKERS
    per_w = n_chunks * CHUNK
    mesh = plsc.VectorSubcoreMesh(
        core_axis_name="c", subcore_axis_name="s", num_cores=2, num_subcores=16
    )
    run = pl.kernel(
        functools.partial(_embed_kernel_body, n_chunks),
        out_type=jax.ShapeDtypeStruct(
            (t, EMBED_DIM // 8, b // CHUNK, 8, CHUNK), jnp.float32
        ),
        mesh=mesh,
        scratch_types=[
            pltpu.VMEM((per_w,), jnp.int32),
            pltpu.VMEM((VOCAB_ROWS, EMBED_DIM), jnp.float32),
            pltpu.VMEM((VOCAB_ROWS * SKEW,), jnp.float32),
            pltpu.VMEM((2, EMBED_DIM, CHUNK), jnp.float32),
            pltpu.SemaphoreType.DMA((2,)),
            pltpu.SemaphoreType.DMA,
        ],
        compiler_params=pltpu.CompilerParams(
            use_tc_tiling_on_sc=False, needs_layout_passes=False
        ),
    )
    return run(idx_tmajor, emb)


def kernel(x, emb):
    b, t = x.shape
    assert emb.shape == (VOCAB_ROWS, EMBED_DIM)
    idx_tmajor = x.T.reshape(-1).astype(jnp.int32)
    y5 = _embed(idx_tmajor, emb, b, t)
    return y5.transpose(2, 4, 0, 1, 3).reshape(b, t, EMBED_DIM)
